# Initial kernel scaffold; baseline (speedup 1.0000x reference)
import jax
import jax.numpy as jnp
from jax.experimental import pallas as pl


def kernel(geometric_features, pos_x_table, pos_y_table, width_table, height_table, proj_w, proj_b, fusion_w, fusion_b):
    raise NotImplementedError("write your pallas kernel here")


# speedup vs baseline: 1.5361x; 1.5361x over previous
import functools

import jax
import jax.numpy as jnp
from jax import lax
from jax.experimental import pallas as pl
from jax.experimental.pallas import tpu as pltpu
from jax.experimental.pallas import tpu_sc as plsc

MAXP = 1000
D = 768
D4 = 192
NC, NS = 2, 16
NW = NC * NS
CT = 16


def _prep_body(px, py, pw, ph, fw, pjw8, pjb, fb, tall_ref, wc_ref):
    w2 = fw[pl.ds(D, D), :]
    b_tot = jnp.dot(pjb[...], w2, preferred_element_type=jnp.float32) + fb[...]
    tabs = (px, py, pw, ph)
    for k in range(4):
        w1k = fw[pl.ds(k * D4, D4), :]
        tk = jnp.dot(tabs[k][...], w1k, preferred_element_type=jnp.float32)
        if k == 0:
            tk = tk + b_tot
        tall_ref[pl.ds(k * MAXP, MAXP), :] = tk
    wc_ref[...] = jnp.dot(pjw8[...], w2, preferred_element_type=jnp.float32)


def _cont_body(gf8_ref, wc_ref, g_ref, out_ref):
    out_ref[...] = g_ref[...] + jnp.dot(
        gf8_ref[...], wc_ref[...], preferred_element_type=jnp.float32)


def _make_gather_kernel(T):
    TPW = T // NW
    NCHUNK = TPW // CT

    mesh = plsc.VectorSubcoreMesh(
        core_axis_name="c", subcore_axis_name="s",
        num_cores=NC, num_subcores=NS)

    @functools.partial(
        pl.kernel, mesh=mesh,
        out_type=jax.ShapeDtypeStruct((T, D), jnp.float32),
        scratch_types=[
            pltpu.VMEM((4 * CT,), jnp.float32),
            pltpu.VMEM((4 * CT,), jnp.int32),
            pltpu.VMEM((4 * CT, D), jnp.float32),
            pltpu.VMEM((CT, D), jnp.float32),
            pltpu.SemaphoreType.DMA,
        ],
    )
    def gather_sum(tall_hbm, gft_hbm, out_hbm, gfc_v, idx_v, gat_v, acc_v, sem):
        wid = lax.axis_index("s") * NC + lax.axis_index("c")
        base0 = wid * TPW

        def chunk(ci, carry):
            base = base0 + ci * CT
            for k in range(4):
                pltpu.sync_copy(gft_hbm.at[k, pl.ds(base, CT)],
                                gfc_v.at[pl.ds(k * CT, CT)])
            for j in range(4 * CT // 16):
                k = j // (CT // 16)
                v = gfc_v[pl.ds(j * 16, 16)]
                iv = jnp.clip(v.astype(jnp.int32), 0, MAXP - 1) + k * MAXP
                idx_v[pl.ds(j * 16, 16)] = iv
            pltpu.async_copy(tall_hbm.at[idx_v], gat_v, sem).wait()

            def tok(t, c2):
                for r in range(D // 16):
                    s = ((gat_v[0 * CT + t, pl.ds(r * 16, 16)]
                          + gat_v[1 * CT + t, pl.ds(r * 16, 16)])
                         + (gat_v[2 * CT + t, pl.ds(r * 16, 16)]
                            + gat_v[3 * CT + t, pl.ds(r * 16, 16)]))
                    acc_v[t, pl.ds(r * 16, 16)] = s
                return c2

            lax.fori_loop(0, CT, tok, 0)
            pltpu.sync_copy(acc_v, out_hbm.at[pl.ds(base, CT)])
            return carry

        lax.fori_loop(0, NCHUNK, chunk, 0)

    return gather_sum


def kernel(geometric_features, pos_x_table, pos_y_table, width_table,
           height_table, proj_w, proj_b, fusion_w, fusion_b):
    B, N, F = geometric_features.shape
    T = B * N
    gff = geometric_features.reshape(T, F)
    gf8 = jnp.concatenate(
        [gff, jnp.zeros((T, 8 - F), dtype=gff.dtype)], axis=1)
    gft4 = gff[:, :4].T
    pjw8 = jnp.concatenate(
        [proj_w, jnp.zeros((8 - F, D), dtype=proj_w.dtype)], axis=0)
    pjb2 = proj_b.reshape(1, D)
    fb2 = fusion_b.reshape(1, D)

    tall, wc = pl.pallas_call(
        _prep_body,
        out_shape=[jax.ShapeDtypeStruct((4 * MAXP, D), jnp.float32),
                   jax.ShapeDtypeStruct((8, D), jnp.float32)],
    )(pos_x_table, pos_y_table, width_table, height_table,
      fusion_w, pjw8, pjb2, fb2)

    g = _make_gather_kernel(T)(tall, gft4)

    BM = 2048
    out = pl.pallas_call(
        _cont_body,
        grid=(T // BM,),
        in_specs=[pl.BlockSpec((BM, 8), lambda i: (i, 0)),
                  pl.BlockSpec((8, D), lambda i: (0, 0)),
                  pl.BlockSpec((BM, D), lambda i: (i, 0))],
        out_specs=pl.BlockSpec((BM, D), lambda i: (i, 0)),
        out_shape=jax.ShapeDtypeStruct((T, D), jnp.float32),
    )(gf8, wc, g)

    return out.reshape(B, N, D)
